# baseline (speedup 1.0000x reference)
import functools

import jax
import jax.numpy as jnp
from jax import lax
from jax.experimental import pallas as pl
from jax.experimental.pallas import tpu as pltpu
from jax.experimental.pallas import tpu_sc as plsc

B, G, D = 16, 20000, 128
S, L = 8, 64
SET_STRIDE = 100
LANES = 16
DH = D // 2
NCH = DH // LANES
NUM_CORES = 2
NUM_SUBCORES = 16
NW = NUM_CORES * NUM_SUBCORES
BP = 8
UNROLL = 8


def _agg_body(gene_hbm, attn_hbm, out_hbm, attn_v, gene_v, out_v, acc_v,
              sem_a, sem_g0, sem_g1, sem_o):
    cid = lax.axis_index("c")
    sid = lax.axis_index("s")
    wid = sid * NUM_CORES + cid
    set_id = wid % S
    half = (wid // S) % 2
    b_base = (wid // (2 * S)) * BP
    col0 = half * DH

    row0 = set_id * SET_STRIDE
    LH = L // 2
    cp_a = pltpu.async_copy(attn_hbm.at[set_id, :, pl.ds(col0, DH)],
                            attn_v, sem_a)
    cp_g0 = pltpu.async_copy(
        gene_hbm.at[pl.ds(b_base, BP), pl.ds(row0, LH), pl.ds(col0, DH)],
        gene_v.at[:, pl.ds(0, LH)], sem_g0)
    cp_g1 = pltpu.async_copy(
        gene_hbm.at[pl.ds(b_base, BP), pl.ds(row0 + LH, LH), pl.ds(col0, DH)],
        gene_v.at[:, pl.ds(LH, LH)], sem_g1)
    cp_a.wait()
    cp_g0.wait()

    def acc_chunk(l_base, flush):
        def chunk_body(c, _):
            o = c * LANES

            def l_body(lu, carry):
                d = carry[0]
                a = list(carry[1:])
                for u in range(UNROLL):
                    l = l_base + lu * UNROLL + u
                    e = jnp.exp(attn_v[l, pl.ds(o, LANES)])
                    d = d + e
                    for b in range(BP):
                        a[b] = a[b] + e * gene_v[b, l, pl.ds(o, LANES)]
                return (d, *a)

            if l_base == 0:
                init = (jnp.zeros((LANES,), jnp.float32),) * (BP + 1)
            else:
                init = tuple(acc_v[b, pl.ds(o, LANES)]
                             for b in range(BP + 1))
            carry = lax.fori_loop(0, LH // UNROLL, l_body, init)
            if not flush:
                for b in range(BP + 1):
                    acc_v[b, pl.ds(o, LANES)] = carry[b]
            else:
                r = 1.0 / carry[0]
                for b in range(BP):
                    out_v[b, pl.ds(o, LANES)] = carry[1 + b] * r
            return 0

        lax.fori_loop(0, NCH, chunk_body, 0)

    acc_chunk(0, False)
    cp_g1.wait()
    acc_chunk(LH, True)

    pltpu.async_copy(out_v,
                     out_hbm.at[pl.ds(b_base, BP), set_id, pl.ds(col0, DH)],
                     sem_o).wait()


@functools.lru_cache(maxsize=None)
def _build_agg():
    return pl.kernel(
        _agg_body,
        out_type=jax.ShapeDtypeStruct((B, S, D), jnp.float32),
        mesh=plsc.VectorSubcoreMesh(core_axis_name="c", subcore_axis_name="s",
                                    num_cores=NUM_CORES,
                                    num_subcores=NUM_SUBCORES),
        scratch_types=[
            pltpu.VMEM((L, DH), jnp.float32),
            pltpu.VMEM((BP, L, DH), jnp.float32),
            pltpu.VMEM((BP, DH), jnp.float32),
            pltpu.VMEM((BP + 1, DH), jnp.float32),
            pltpu.SemaphoreType.DMA,
            pltpu.SemaphoreType.DMA,
            pltpu.SemaphoreType.DMA,
            pltpu.SemaphoreType.DMA,
        ],
        compiler_params=pltpu.CompilerParams(use_tc_tiling_on_sc=False,
                                             skip_device_barrier=True),
    )


def kernel(gene_features, attn_weights):
    return _build_agg()(gene_features, attn_weights)
